# baseline (speedup 1.0000x reference)
import functools

import jax
import jax.numpy as jnp
from jax import lax
from jax.experimental import pallas as pl
from jax.experimental.pallas import tpu as pltpu
from jax.experimental.pallas import tpu_sc as plsc

M_TILE = 1000

_NS = 16
_NC = 2
_K = 128


def _leaky(x):
    return jnp.where(x > 0, x, 0.01 * x)


def _blockify(x, nb):
    n = x.shape[0]
    return jnp.moveaxis(x.reshape(n, nb, 128), 1, 0)


def _gc_body(agg_ref, h_ref, wrel_ref, wroot_ref, b_ref, o_ref):
    nb_in = agg_ref.shape[0]
    aggc = jnp.concatenate([agg_ref[i] for i in range(nb_in)], axis=1)
    hc = jnp.concatenate([h_ref[i] for i in range(nb_in)], axis=1)
    pre = (jnp.dot(aggc, wrel_ref[0], preferred_element_type=jnp.float32)
           + b_ref[0]) + jnp.dot(hc, wroot_ref[0],
                                 preferred_element_type=jnp.float32)
    o_ref[0] = _leaky(pre)


def _gc_layer(agg, h, w_rel, b_rel, w_root, nb_out):
    nb_in, n_pad_, _ = agg.shape
    n = h.shape[1]
    d_in, d_out = w_rel.shape
    nm = n // M_TILE
    wrel_blk = jnp.moveaxis(w_rel.reshape(d_in, nb_out, 128), 1, 0)
    wroot_blk = jnp.moveaxis(w_root.reshape(d_in, nb_out, 128), 1, 0)
    b_blk = b_rel.reshape(nb_out, 1, 128)
    return pl.pallas_call(
        _gc_body,
        grid=(nm, nb_out),
        in_specs=[
            pl.BlockSpec((nb_in, M_TILE, 128), lambda m, b: (0, m, 0)),
            pl.BlockSpec((nb_in, M_TILE, 128), lambda m, b: (0, m, 0)),
            pl.BlockSpec((1, d_in, 128), lambda m, b: (b, 0, 0)),
            pl.BlockSpec((1, d_in, 128), lambda m, b: (b, 0, 0)),
            pl.BlockSpec((1, 1, 128), lambda m, b: (b, 0, 0)),
        ],
        out_specs=pl.BlockSpec((1, M_TILE, 128), lambda m, b: (b, m, 0)),
        out_shape=jax.ShapeDtypeStruct((nb_out, n, 128), jnp.float32),
    )(agg, h, wrel_blk, wroot_blk, b_blk)


def _final_body(agg_ref, h_ref, wrel_ref, b3_ref, wroot_ref, wh_ref, bh_ref,
                o_ref):
    agg3 = agg_ref[0] + agg_ref[1]
    h2 = h_ref[0]
    pre = (jnp.dot(agg3, wrel_ref[...], preferred_element_type=jnp.float32)
           + b3_ref[...]) + jnp.dot(h2, wroot_ref[...],
                                    preferred_element_type=jnp.float32)
    h3 = _leaky(pre)
    y = jnp.dot(h3, wh_ref[...], preferred_element_type=jnp.float32) + bh_ref[...]
    pos = y[:, 0:3]
    rot = y[:, 3:7]
    norm = jnp.maximum(
        jnp.sqrt(jnp.sum(rot * rot, axis=1, keepdims=True)), 1e-12)
    o_ref[...] = jnp.concatenate(
        [pos, rot / norm, jnp.zeros_like(y[:, 7:8])], axis=1)


def _final(agg, h2, w3_rel, b3, w3_root, w_pos, b_pos, w_rot, b_rot):
    n = h2.shape[1]
    nm = n // M_TILE
    w_head = jnp.pad(jnp.concatenate([w_pos, w_rot], axis=1), ((0, 0), (0, 1)))
    b_head = jnp.pad(jnp.concatenate([b_pos, b_rot]), (0, 1))
    return pl.pallas_call(
        _final_body,
        grid=(nm,),
        in_specs=[
            pl.BlockSpec((2, M_TILE, 128), lambda m: (0, m, 0)),
            pl.BlockSpec((1, M_TILE, 128), lambda m: (0, m, 0)),
            pl.BlockSpec((128, 64), lambda m: (0, 0)),
            pl.BlockSpec((1, 64), lambda m: (0, 0)),
            pl.BlockSpec((128, 64), lambda m: (0, 0)),
            pl.BlockSpec((64, 8), lambda m: (0, 0)),
            pl.BlockSpec((1, 8), lambda m: (0, 0)),
        ],
        out_specs=pl.BlockSpec((M_TILE, 8), lambda m: (m, 0)),
        out_shape=jax.ShapeDtypeStruct((n, 8), jnp.float32),
    )(agg, h2, w3_rel, b3.reshape(1, 64), w3_root, w_head,
      b_head.reshape(1, 8))


def _sc_segsum_call(p_blocked, idx, zeros, n_pad, esplit, c_round, n_tasks):
    bpc = n_tasks // _NC
    stripe = n_pad // _NS
    w = p_blocked.shape[2]

    def body(p_ref, idx_ref, zeros_ref, out_ref,
             ibufs, rbufs, acc, semi, semg, sems):
        c = lax.axis_index("c")
        s = lax.axis_index("s")
        row0 = s * stripe
        for b in range(bpc):
            t = c * bpc + b
            blk = 0 if esplit == 2 else t
            base = t * c_round if esplit == 2 else 0
            pltpu.sync_copy(zeros_ref, acc.at[pl.ds(row0, stripe)])
            plsc.subcore_barrier()

            d0, d1 = c_total_dim - 2, c_total_dim - 1
            pltpu.sync_copy(idx_ref.at[s, d0], ibufs.at[2])
            pltpu.sync_copy(idx_ref.at[s, d1], ibufs.at[3])
            pltpu.async_copy(rbufs.at[0], acc.at[ibufs.at[3].at[1]],
                             sems.at[0], add=True)
            pltpu.async_copy(rbufs.at[1], acc.at[ibufs.at[2].at[1]],
                             sems.at[1], add=True)
            pltpu.sync_copy(idx_ref.at[s, base], ibufs.at[0])
            pltpu.async_copy(idx_ref.at[s, base + 1], ibufs.at[1],
                             semi.at[1])
            pltpu.async_copy(p_ref.at[blk].at[ibufs.at[0].at[0]],
                             rbufs.at[0], semg.at[0])

            def quad(i, _):
                for j in range(4):
                    k = 4 * i + j
                    r, r1 = j % 2, (j + 1) % 2
                    q, q1, q2 = j % 4, (j + 1) % 4, (j + 2) % 4
                    pltpu.make_async_copy(
                        idx_ref.at[s, base + k + 1], ibufs.at[q1],
                        semi.at[q1]).wait()
                    pltpu.make_async_copy(
                        rbufs.at[r1], acc.at[pl.ds(0, _K)],
                        sems.at[r1]).wait()
                    pltpu.async_copy(idx_ref.at[s, base + k + 2],
                                     ibufs.at[q2], semi.at[q2])
                    pltpu.make_async_copy(
                        p_ref.at[blk].at[ibufs.at[q].at[0]], rbufs.at[r],
                        semg.at[r]).wait()
                    pltpu.async_copy(p_ref.at[blk].at[ibufs.at[q1].at[0]],
                                     rbufs.at[r1], semg.at[r1])
                    pltpu.async_copy(rbufs.at[r], acc.at[ibufs.at[q].at[1]],
                                     sems.at[r], add=True)
                return 0

            lax.fori_loop(0, c_round // 4, quad, 0)
            cm = c_round % 4
            pltpu.make_async_copy(
                p_ref.at[blk].at[ibufs.at[cm].at[0]],
                rbufs.at[c_round % 2], semg.at[c_round % 2]).wait()
            pltpu.make_async_copy(
                rbufs.at[(c_round + 1) % 2], acc.at[pl.ds(0, _K)],
                sems.at[(c_round + 1) % 2]).wait()
            pltpu.make_async_copy(
                idx_ref.at[s, base + c_round + 1],
                ibufs.at[(c_round + 1) % 4], semi.at[(c_round + 1) % 4]).wait()
            plsc.subcore_barrier()
            pltpu.sync_copy(acc.at[pl.ds(row0, stripe)],
                            out_ref.at[t].at[pl.ds(row0, stripe)])

    c_total_dim = idx.shape[1]
    mesh = plsc.VectorSubcoreMesh(core_axis_name="c", subcore_axis_name="s")
    return pl.kernel(
        body,
        out_type=jax.ShapeDtypeStruct((n_tasks, n_pad, w), jnp.float32),
        mesh=mesh,
        scratch_types=[
            pltpu.VMEM((4, 2, _K), jnp.int32),
            pltpu.VMEM((2, _K, w), jnp.float32),
            pltpu.VMEM_SHARED((n_pad, w), jnp.float32),
            pltpu.SemaphoreType.DMA((4,)),
            pltpu.SemaphoreType.DMA((2,)),
            pltpu.SemaphoreType.DMA((2,)),
        ],
    )(p_blocked, idx, zeros)


def _edge_index_chunks(src, dst, n, n_pad):
    e = src.shape[0]
    per_tile = e // _NS
    c_total = -(-per_tile // _K)
    if c_total % 2:
        c_total += 1
    pad = c_total * _K - per_tile
    spare = n_pad - n
    tile_ids = jnp.arange(_NS, dtype=jnp.int32)[:, None]
    pad_dst = n + (tile_ids * 7 + jnp.arange(pad, dtype=jnp.int32)) % spare
    src_t = jnp.pad(src.reshape(_NS, per_tile), ((0, 0), (0, pad)))
    dst_t = jnp.concatenate([dst.reshape(_NS, per_tile), pad_dst], axis=1)
    idx = jnp.stack([src_t.reshape(_NS, c_total, _K),
                     dst_t.reshape(_NS, c_total, _K)], axis=2)
    dummy_dst = (n + (tile_ids * 7 + jnp.arange(2 * _K, dtype=jnp.int32))
                 % spare).reshape(_NS, 2, _K)
    dummy = jnp.stack([jnp.zeros((_NS, 2, _K), jnp.int32), dummy_dst],
                      axis=2)
    return jnp.concatenate([idx, dummy], axis=1), c_total


def kernel(x, edge_index, W1_rel, b1_rel, W1_root, W2_rel, b2_rel, W2_root,
           W3_rel, b3_rel, W3_root, W_pos, b_pos, W_rot, b_rot):
    n = x.shape[0]
    src, dst = edge_index[0], edge_index[1]
    stripe = -(-n // (_NS * 8)) * 8
    n_pad = stripe * _NS
    idx, c_total = _edge_index_chunks(src, dst, n, n_pad)
    zeros = jnp.zeros((stripe, 128), jnp.float32)

    x_blk = _blockify(x, 8)
    agg1 = _sc_segsum_call(x_blk, idx, zeros, n_pad, 1, c_total, 8)
    h1 = _gc_layer(agg1, x_blk, W1_rel, b1_rel, W1_root, 4)
    agg2 = _sc_segsum_call(h1, idx, zeros, n_pad, 1, c_total, 4)
    h2 = _gc_layer(agg2, h1, W2_rel, b2_rel, W2_root, 1)
    agg3 = _sc_segsum_call(h2, idx, zeros, n_pad, 2, c_total // 2, 2)
    return _final(agg3, h2, W3_rel, b3_rel, W3_root,
                  W_pos, b_pos, W_rot, b_rot)[:, :7]


# speedup vs baseline: 1.0008x; 1.0008x over previous
import functools

import jax
import jax.numpy as jnp
from jax import lax
from jax.experimental import pallas as pl
from jax.experimental.pallas import tpu as pltpu
from jax.experimental.pallas import tpu_sc as plsc

M_TILE = 1000

_NS = 16
_NC = 2
_K = 128


def _leaky(x):
    return jnp.where(x > 0, x, 0.01 * x)


def _blockify(x, nb):
    n = x.shape[0]
    return jnp.moveaxis(x.reshape(n, nb, 128), 1, 0)


def _gc_body(agg_ref, h_ref, wrel_ref, wroot_ref, b_ref, o_ref):
    nb_in = agg_ref.shape[0]
    aggc = jnp.concatenate([agg_ref[i] for i in range(nb_in)], axis=1)
    hc = jnp.concatenate([h_ref[i] for i in range(nb_in)], axis=1)
    pre = (jnp.dot(aggc, wrel_ref[0], preferred_element_type=jnp.float32)
           + b_ref[0]) + jnp.dot(hc, wroot_ref[0],
                                 preferred_element_type=jnp.float32)
    o_ref[0] = _leaky(pre)


def _gc_layer(agg, h, w_rel, b_rel, w_root, nb_out):
    nb_in, n_pad_, _ = agg.shape
    n = h.shape[1]
    d_in, d_out = w_rel.shape
    nm = n // M_TILE
    wrel_blk = jnp.moveaxis(w_rel.reshape(d_in, nb_out, 128), 1, 0)
    wroot_blk = jnp.moveaxis(w_root.reshape(d_in, nb_out, 128), 1, 0)
    b_blk = b_rel.reshape(nb_out, 1, 128)
    return pl.pallas_call(
        _gc_body,
        grid=(nm, nb_out),
        in_specs=[
            pl.BlockSpec((nb_in, M_TILE, 128), lambda m, b: (0, m, 0)),
            pl.BlockSpec((nb_in, M_TILE, 128), lambda m, b: (0, m, 0)),
            pl.BlockSpec((1, d_in, 128), lambda m, b: (b, 0, 0)),
            pl.BlockSpec((1, d_in, 128), lambda m, b: (b, 0, 0)),
            pl.BlockSpec((1, 1, 128), lambda m, b: (b, 0, 0)),
        ],
        out_specs=pl.BlockSpec((1, M_TILE, 128), lambda m, b: (b, m, 0)),
        out_shape=jax.ShapeDtypeStruct((nb_out, n, 128), jnp.float32),
    )(agg, h, wrel_blk, wroot_blk, b_blk)


def _final_body(agg_ref, h_ref, wrel_ref, b3_ref, wroot_ref, wh_ref, bh_ref,
                o_ref):
    agg3 = agg_ref[0] + agg_ref[1]
    h2 = h_ref[0]
    pre = (jnp.dot(agg3, wrel_ref[...], preferred_element_type=jnp.float32)
           + b3_ref[...]) + jnp.dot(h2, wroot_ref[...],
                                    preferred_element_type=jnp.float32)
    h3 = _leaky(pre)
    y = jnp.dot(h3, wh_ref[...], preferred_element_type=jnp.float32) + bh_ref[...]
    pos = y[:, 0:3]
    rot = y[:, 3:7]
    norm = jnp.maximum(
        jnp.sqrt(jnp.sum(rot * rot, axis=1, keepdims=True)), 1e-12)
    o_ref[...] = jnp.concatenate(
        [pos, rot / norm, jnp.zeros_like(y[:, 7:8])], axis=1)


def _final(agg, h2, w3_rel, b3, w3_root, w_pos, b_pos, w_rot, b_rot):
    n = h2.shape[1]
    nm = n // M_TILE
    w_head = jnp.pad(jnp.concatenate([w_pos, w_rot], axis=1), ((0, 0), (0, 1)))
    b_head = jnp.pad(jnp.concatenate([b_pos, b_rot]), (0, 1))
    return pl.pallas_call(
        _final_body,
        grid=(nm,),
        in_specs=[
            pl.BlockSpec((2, M_TILE, 128), lambda m: (0, m, 0)),
            pl.BlockSpec((1, M_TILE, 128), lambda m: (0, m, 0)),
            pl.BlockSpec((128, 64), lambda m: (0, 0)),
            pl.BlockSpec((1, 64), lambda m: (0, 0)),
            pl.BlockSpec((128, 64), lambda m: (0, 0)),
            pl.BlockSpec((64, 8), lambda m: (0, 0)),
            pl.BlockSpec((1, 8), lambda m: (0, 0)),
        ],
        out_specs=pl.BlockSpec((M_TILE, 8), lambda m: (m, 0)),
        out_shape=jax.ShapeDtypeStruct((n, 8), jnp.float32),
    )(agg, h2, w3_rel, b3.reshape(1, 64), w3_root, w_head,
      b_head.reshape(1, 8))


def _sc_segsum_call(p_blocked, idx, zeros, n_pad, esplit, c_round, n_tasks):
    bpc = n_tasks // _NC
    stripe = n_pad // _NS
    w = p_blocked.shape[2]

    def body(p_ref, idx_ref, zeros_ref, out_ref,
             ibufs, rbufs, acc, semi, semg, sems):
        c = lax.axis_index("c")
        s = lax.axis_index("s")
        row0 = s * stripe
        for b in range(bpc):
            t = c * bpc + b
            blk = t
            base = t * c_round if esplit == 2 else 0
            pltpu.sync_copy(zeros_ref, acc.at[pl.ds(row0, stripe)])
            plsc.subcore_barrier()

            d0, d1 = c_total_dim - 2, c_total_dim - 1
            pltpu.sync_copy(idx_ref.at[s, d0], ibufs.at[2])
            pltpu.sync_copy(idx_ref.at[s, d1], ibufs.at[3])
            pltpu.async_copy(rbufs.at[0], acc.at[ibufs.at[3].at[1]],
                             sems.at[0], add=True)
            pltpu.async_copy(rbufs.at[1], acc.at[ibufs.at[2].at[1]],
                             sems.at[1], add=True)
            pltpu.sync_copy(idx_ref.at[s, base], ibufs.at[0])
            pltpu.async_copy(idx_ref.at[s, base + 1], ibufs.at[1],
                             semi.at[1])
            pltpu.async_copy(p_ref.at[blk].at[ibufs.at[0].at[0]],
                             rbufs.at[0], semg.at[0])

            def quad(i, _):
                for j in range(4):
                    k = 4 * i + j
                    r, r1 = j % 2, (j + 1) % 2
                    q, q1, q2 = j % 4, (j + 1) % 4, (j + 2) % 4
                    pltpu.make_async_copy(
                        idx_ref.at[s, base + k + 1], ibufs.at[q1],
                        semi.at[q1]).wait()
                    pltpu.make_async_copy(
                        rbufs.at[r1], acc.at[pl.ds(0, _K)],
                        sems.at[r1]).wait()
                    pltpu.async_copy(idx_ref.at[s, base + k + 2],
                                     ibufs.at[q2], semi.at[q2])
                    pltpu.make_async_copy(
                        p_ref.at[blk].at[ibufs.at[q].at[0]], rbufs.at[r],
                        semg.at[r]).wait()
                    pltpu.async_copy(p_ref.at[blk].at[ibufs.at[q1].at[0]],
                                     rbufs.at[r1], semg.at[r1])
                    pltpu.async_copy(rbufs.at[r], acc.at[ibufs.at[q].at[1]],
                                     sems.at[r], add=True)
                return 0

            lax.fori_loop(0, c_round // 4, quad, 0)
            cm = c_round % 4
            pltpu.make_async_copy(
                p_ref.at[blk].at[ibufs.at[cm].at[0]],
                rbufs.at[c_round % 2], semg.at[c_round % 2]).wait()
            pltpu.make_async_copy(
                rbufs.at[(c_round + 1) % 2], acc.at[pl.ds(0, _K)],
                sems.at[(c_round + 1) % 2]).wait()
            pltpu.make_async_copy(
                idx_ref.at[s, base + c_round + 1],
                ibufs.at[(c_round + 1) % 4], semi.at[(c_round + 1) % 4]).wait()
            plsc.subcore_barrier()
            pltpu.sync_copy(acc.at[pl.ds(row0, stripe)],
                            out_ref.at[t].at[pl.ds(row0, stripe)])

    c_total_dim = idx.shape[1]
    mesh = plsc.VectorSubcoreMesh(core_axis_name="c", subcore_axis_name="s")
    return pl.kernel(
        body,
        out_type=jax.ShapeDtypeStruct((n_tasks, n_pad, w), jnp.float32),
        mesh=mesh,
        scratch_types=[
            pltpu.VMEM((4, 2, _K), jnp.int32),
            pltpu.VMEM((2, _K, w), jnp.float32),
            pltpu.VMEM_SHARED((n_pad, w), jnp.float32),
            pltpu.SemaphoreType.DMA((4,)),
            pltpu.SemaphoreType.DMA((2,)),
            pltpu.SemaphoreType.DMA((2,)),
        ],
    )(p_blocked, idx, zeros)


def _edge_index_chunks(src, dst, n, n_pad):
    e = src.shape[0]
    per_tile = e // _NS
    c_total = -(-per_tile // _K)
    if c_total % 2:
        c_total += 1
    pad = c_total * _K - per_tile
    spare = n_pad - n
    tile_ids = jnp.arange(_NS, dtype=jnp.int32)[:, None]
    pad_dst = n + (tile_ids * 7 + jnp.arange(pad, dtype=jnp.int32)) % spare
    src_t = jnp.pad(src.reshape(_NS, per_tile), ((0, 0), (0, pad)))
    dst_t = jnp.concatenate([dst.reshape(_NS, per_tile), pad_dst], axis=1)
    idx = jnp.stack([src_t.reshape(_NS, c_total, _K),
                     dst_t.reshape(_NS, c_total, _K)], axis=2)
    dummy_dst = (n + (tile_ids * 7 + jnp.arange(2 * _K, dtype=jnp.int32))
                 % spare).reshape(_NS, 2, _K)
    dummy = jnp.stack([jnp.zeros((_NS, 2, _K), jnp.int32), dummy_dst],
                      axis=2)
    return jnp.concatenate([idx, dummy], axis=1), c_total


def kernel(x, edge_index, W1_rel, b1_rel, W1_root, W2_rel, b2_rel, W2_root,
           W3_rel, b3_rel, W3_root, W_pos, b_pos, W_rot, b_rot):
    n = x.shape[0]
    src, dst = edge_index[0], edge_index[1]
    stripe = -(-n // (_NS * 8)) * 8
    n_pad = stripe * _NS
    idx, c_total = _edge_index_chunks(src, dst, n, n_pad)
    zeros = jnp.zeros((stripe, 128), jnp.float32)

    x_blk = _blockify(x, 8)
    agg1 = _sc_segsum_call(x_blk, idx, zeros, n_pad, 1, c_total, 8)
    h1 = _gc_layer(agg1, x_blk, W1_rel, b1_rel, W1_root, 4)
    agg2 = _sc_segsum_call(h1, idx, zeros, n_pad, 1, c_total, 4)
    h2 = _gc_layer(agg2, h1, W2_rel, b2_rel, W2_root, 1)
    h2_dup = jnp.concatenate([h2, h2], axis=0)
    agg3 = _sc_segsum_call(h2_dup, idx, zeros, n_pad, 2, c_total // 2, 2)
    return _final(agg3, h2, W3_rel, b3_rel, W3_root,
                  W_pos, b_pos, W_rot, b_rot)[:, :7]


# speedup vs baseline: 1.0060x; 1.0051x over previous
import functools

import jax
import jax.numpy as jnp
from jax import lax
from jax.experimental import pallas as pl
from jax.experimental.pallas import tpu as pltpu
from jax.experimental.pallas import tpu_sc as plsc

M_TILE = 1000

_NS = 16
_NC = 2
_K = 128


def _leaky(x):
    return jnp.where(x > 0, x, 0.01 * x)


def _blockify(x, nb):
    n = x.shape[0]
    return jnp.moveaxis(x.reshape(n, nb, 128), 1, 0)


def _root_body(h_ref, wroot_ref, o_ref):
    nb_in = h_ref.shape[0]
    hc = jnp.concatenate([h_ref[i] for i in range(nb_in)], axis=1)
    o_ref[0] = jnp.dot(hc, wroot_ref[0], preferred_element_type=jnp.float32)


def _root_mm(h, w_root, nb_out):
    nb_in, n, _ = h.shape
    d_in = w_root.shape[0]
    nm = n // M_TILE
    wroot_blk = jnp.moveaxis(w_root.reshape(d_in, nb_out, 128), 1, 0)
    return pl.pallas_call(
        _root_body,
        grid=(nm, nb_out),
        in_specs=[
            pl.BlockSpec((nb_in, M_TILE, 128), lambda m, b: (0, m, 0)),
            pl.BlockSpec((1, d_in, 128), lambda m, b: (b, 0, 0)),
        ],
        out_specs=pl.BlockSpec((1, M_TILE, 128), lambda m, b: (b, m, 0)),
        out_shape=jax.ShapeDtypeStruct((nb_out, n, 128), jnp.float32),
    )(h, wroot_blk)


def _gc_body(agg_ref, r_ref, wrel_ref, b_ref, o_ref):
    nb_in = agg_ref.shape[0]
    aggc = jnp.concatenate([agg_ref[i] for i in range(nb_in)], axis=1)
    pre = (jnp.dot(aggc, wrel_ref[0], preferred_element_type=jnp.float32)
           + b_ref[0]) + r_ref[0]
    o_ref[0] = _leaky(pre)


def _gc_layer(agg, r, w_rel, b_rel, nb_out):
    nb_in, n_pad_, _ = agg.shape
    n = r.shape[1]
    d_in = w_rel.shape[0]
    nm = n // M_TILE
    wrel_blk = jnp.moveaxis(w_rel.reshape(d_in, nb_out, 128), 1, 0)
    b_blk = b_rel.reshape(nb_out, 1, 128)
    return pl.pallas_call(
        _gc_body,
        grid=(nm, nb_out),
        in_specs=[
            pl.BlockSpec((nb_in, M_TILE, 128), lambda m, b: (0, m, 0)),
            pl.BlockSpec((1, M_TILE, 128), lambda m, b: (b, m, 0)),
            pl.BlockSpec((1, d_in, 128), lambda m, b: (b, 0, 0)),
            pl.BlockSpec((1, 1, 128), lambda m, b: (b, 0, 0)),
        ],
        out_specs=pl.BlockSpec((1, M_TILE, 128), lambda m, b: (b, m, 0)),
        out_shape=jax.ShapeDtypeStruct((nb_out, n, 128), jnp.float32),
    )(agg, r, wrel_blk, b_blk)


def _final_body(agg_ref, h_ref, wrel_ref, b3_ref, wroot_ref, wh_ref, bh_ref,
                o_ref):
    agg3 = agg_ref[0] + agg_ref[1]
    h2 = h_ref[0]
    pre = (jnp.dot(agg3, wrel_ref[...], preferred_element_type=jnp.float32)
           + b3_ref[...]) + jnp.dot(h2, wroot_ref[...],
                                    preferred_element_type=jnp.float32)
    h3 = _leaky(pre)
    y = jnp.dot(h3, wh_ref[...], preferred_element_type=jnp.float32) + bh_ref[...]
    pos = y[:, 0:3]
    rot = y[:, 3:7]
    norm = jnp.maximum(
        jnp.sqrt(jnp.sum(rot * rot, axis=1, keepdims=True)), 1e-12)
    o_ref[...] = jnp.concatenate(
        [pos, rot / norm, jnp.zeros_like(y[:, 7:8])], axis=1)


def _final(agg, h2, w3_rel, b3, w3_root, w_pos, b_pos, w_rot, b_rot):
    n = h2.shape[1]
    nm = n // M_TILE
    w_head = jnp.pad(jnp.concatenate([w_pos, w_rot], axis=1), ((0, 0), (0, 1)))
    b_head = jnp.pad(jnp.concatenate([b_pos, b_rot]), (0, 1))
    return pl.pallas_call(
        _final_body,
        grid=(nm,),
        in_specs=[
            pl.BlockSpec((2, M_TILE, 128), lambda m: (0, m, 0)),
            pl.BlockSpec((1, M_TILE, 128), lambda m: (0, m, 0)),
            pl.BlockSpec((128, 64), lambda m: (0, 0)),
            pl.BlockSpec((1, 64), lambda m: (0, 0)),
            pl.BlockSpec((128, 64), lambda m: (0, 0)),
            pl.BlockSpec((64, 8), lambda m: (0, 0)),
            pl.BlockSpec((1, 8), lambda m: (0, 0)),
        ],
        out_specs=pl.BlockSpec((M_TILE, 8), lambda m: (m, 0)),
        out_shape=jax.ShapeDtypeStruct((n, 8), jnp.float32),
    )(agg, h2, w3_rel, b3.reshape(1, 64), w3_root, w_head,
      b_head.reshape(1, 8))


def _sc_segsum_call(p_blocked, idx, zeros, n_pad, esplit, c_round, n_tasks):
    bpc = n_tasks // _NC
    stripe = n_pad // _NS
    w = p_blocked.shape[2]

    def body(p_ref, idx_ref, zeros_ref, out_ref,
             ibufs, rbufs, acc, semi, semg, sems):
        c = lax.axis_index("c")
        s = lax.axis_index("s")
        row0 = s * stripe
        for b in range(bpc):
            t = c * bpc + b
            blk = t
            base = t * c_round if esplit == 2 else 0
            pltpu.sync_copy(zeros_ref, acc.at[pl.ds(row0, stripe)])
            plsc.subcore_barrier()

            d0, d1 = c_total_dim - 2, c_total_dim - 1
            pltpu.sync_copy(idx_ref.at[s, d0], ibufs.at[2])
            pltpu.sync_copy(idx_ref.at[s, d1], ibufs.at[3])
            pltpu.async_copy(rbufs.at[0], acc.at[ibufs.at[3].at[1]],
                             sems.at[0], add=True)
            pltpu.async_copy(rbufs.at[1], acc.at[ibufs.at[2].at[1]],
                             sems.at[1], add=True)
            pltpu.sync_copy(idx_ref.at[s, base], ibufs.at[0])
            pltpu.async_copy(idx_ref.at[s, base + 1], ibufs.at[1],
                             semi.at[1])
            pltpu.async_copy(p_ref.at[blk].at[ibufs.at[0].at[0]],
                             rbufs.at[0], semg.at[0])

            def quad(i, _):
                for j in range(4):
                    k = 4 * i + j
                    r, r1 = j % 2, (j + 1) % 2
                    q, q1, q2 = j % 4, (j + 1) % 4, (j + 2) % 4
                    pltpu.make_async_copy(
                        idx_ref.at[s, base + k + 1], ibufs.at[q1],
                        semi.at[q1]).wait()
                    pltpu.make_async_copy(
                        rbufs.at[r1], acc.at[pl.ds(0, _K)],
                        sems.at[r1]).wait()
                    pltpu.async_copy(idx_ref.at[s, base + k + 2],
                                     ibufs.at[q2], semi.at[q2])
                    pltpu.make_async_copy(
                        p_ref.at[blk].at[ibufs.at[q].at[0]], rbufs.at[r],
                        semg.at[r]).wait()
                    pltpu.async_copy(p_ref.at[blk].at[ibufs.at[q1].at[0]],
                                     rbufs.at[r1], semg.at[r1])
                    pltpu.async_copy(rbufs.at[r], acc.at[ibufs.at[q].at[1]],
                                     sems.at[r], add=True)
                return 0

            lax.fori_loop(0, c_round // 4, quad, 0)
            cm = c_round % 4
            pltpu.make_async_copy(
                p_ref.at[blk].at[ibufs.at[cm].at[0]],
                rbufs.at[c_round % 2], semg.at[c_round % 2]).wait()
            pltpu.make_async_copy(
                rbufs.at[(c_round + 1) % 2], acc.at[pl.ds(0, _K)],
                sems.at[(c_round + 1) % 2]).wait()
            pltpu.make_async_copy(
                idx_ref.at[s, base + c_round + 1],
                ibufs.at[(c_round + 1) % 4], semi.at[(c_round + 1) % 4]).wait()
            plsc.subcore_barrier()
            pltpu.sync_copy(acc.at[pl.ds(row0, stripe)],
                            out_ref.at[t].at[pl.ds(row0, stripe)])

    c_total_dim = idx.shape[1]
    mesh = plsc.VectorSubcoreMesh(core_axis_name="c", subcore_axis_name="s")
    return pl.kernel(
        body,
        out_type=jax.ShapeDtypeStruct((n_tasks, n_pad, w), jnp.float32),
        mesh=mesh,
        scratch_types=[
            pltpu.VMEM((4, 2, _K), jnp.int32),
            pltpu.VMEM((2, _K, w), jnp.float32),
            pltpu.VMEM_SHARED((n_pad, w), jnp.float32),
            pltpu.SemaphoreType.DMA((4,)),
            pltpu.SemaphoreType.DMA((2,)),
            pltpu.SemaphoreType.DMA((2,)),
        ],
    )(p_blocked, idx, zeros)


def _edge_index_chunks(src, dst, n, n_pad):
    e = src.shape[0]
    per_tile = e // _NS
    c_total = -(-per_tile // _K)
    if c_total % 2:
        c_total += 1
    pad = c_total * _K - per_tile
    spare = n_pad - n
    tile_ids = jnp.arange(_NS, dtype=jnp.int32)[:, None]
    pad_dst = n + (tile_ids * 7 + jnp.arange(pad, dtype=jnp.int32)) % spare
    src_t = jnp.pad(src.reshape(_NS, per_tile), ((0, 0), (0, pad)))
    dst_t = jnp.concatenate([dst.reshape(_NS, per_tile), pad_dst], axis=1)
    idx = jnp.stack([src_t.reshape(_NS, c_total, _K),
                     dst_t.reshape(_NS, c_total, _K)], axis=2)
    dummy_dst = (n + (tile_ids * 7 + jnp.arange(2 * _K, dtype=jnp.int32))
                 % spare).reshape(_NS, 2, _K)
    dummy = jnp.stack([jnp.zeros((_NS, 2, _K), jnp.int32), dummy_dst],
                      axis=2)
    return jnp.concatenate([idx, dummy], axis=1), c_total


def kernel(x, edge_index, W1_rel, b1_rel, W1_root, W2_rel, b2_rel, W2_root,
           W3_rel, b3_rel, W3_root, W_pos, b_pos, W_rot, b_rot):
    n = x.shape[0]
    src, dst = edge_index[0], edge_index[1]
    stripe = -(-n // (_NS * 8)) * 8
    n_pad = stripe * _NS
    idx, c_total = _edge_index_chunks(src, dst, n, n_pad)
    zeros = jnp.zeros((stripe, 128), jnp.float32)

    x_blk = _blockify(x, 8)
    r1 = _root_mm(x_blk, W1_root, 4)
    agg1 = _sc_segsum_call(x_blk, idx, zeros, n_pad, 1, c_total, 8)
    h1 = _gc_layer(agg1, r1, W1_rel, b1_rel, 4)
    r2 = _root_mm(h1, W2_root, 1)
    agg2 = _sc_segsum_call(h1, idx, zeros, n_pad, 1, c_total, 4)
    h2 = _gc_layer(agg2, r2, W2_rel, b2_rel, 1)
    h2_dup = jnp.concatenate([h2, h2], axis=0)
    agg3 = _sc_segsum_call(h2_dup, idx, zeros, n_pad, 2, c_total // 2, 2)
    return _final(agg3, h2, W3_rel, b3_rel, W3_root,
                  W_pos, b_pos, W_rot, b_rot)[:, :7]
